# Initial kernel scaffold; baseline (speedup 1.0000x reference)
import jax
import jax.numpy as jnp
from jax.experimental import pallas as pl


def kernel(point_set_1, point_set_2, feats, key_points):
    raise NotImplementedError("write your pallas kernel here")


# speedup vs baseline: 1.6353x; 1.6353x over previous
import functools

import jax
import jax.numpy as jnp
from jax import lax
from jax.experimental import pallas as pl
from jax.experimental.pallas import tpu as pltpu
from jax.experimental.pallas import tpu_sc as plsc

_N_OBJ = 8
_N_P = 64
_P2 = 4096
_C = 256
_H = 128
_W = 128
_IMG = 512.0
_SIM_THR = 0.5
_LOSS_WEIGHT = 1.0

_N_KP = _N_OBJ * _N_P
_P_TOTAL = _N_KP + _N_OBJ * _P2
_NW = 32
_PPW = _P_TOTAL // _NW
_CH = 80
_NCHUNK = _PPW // _CH


def _make_sampler():
    mesh = plsc.VectorSubcoreMesh(core_axis_name="c", subcore_axis_name="s")

    @functools.partial(
        pl.kernel,
        mesh=mesh,
        out_type=jax.ShapeDtypeStruct((_P_TOTAL, _C), jnp.float32),
        scratch_types=[
            pltpu.VMEM((_NCHUNK * 4, _CH), jnp.int32),
            pltpu.VMEM((_NCHUNK * 4 * _CH + 16,), jnp.float32),
            pltpu.VMEM((_CH, _C), jnp.float32),
            pltpu.VMEM((_CH, _C), jnp.float32),
            pltpu.VMEM((_CH, _C), jnp.float32),
            pltpu.VMEM((_CH, _C), jnp.float32),
            pltpu.VMEM((_CH, _C), jnp.float32),
            pltpu.SemaphoreType.DMA,
        ],
    )
    def sampler(table_hbm, idx_hbm, w_hbm, out_hbm,
                idx_v, w_v, b0, b1, b2, b3, o_v, sem):
        wid = lax.axis_index("s") * 2 + lax.axis_index("c")
        base = wid * _PPW
        pltpu.sync_copy(idx_hbm.at[wid], idx_v)
        pltpu.sync_copy(w_hbm.at[wid, 0], w_v.at[pl.ds(0, _NCHUNK * 4 * _CH)])

        def chunk_body(k, carry):
            off = base + k * _CH
            cps = [
                pltpu.async_copy(table_hbm.at[idx_v.at[k * 4 + c]], buf, sem)
                for c, buf in enumerate((b0, b1, b2, b3))
            ]
            for cp in cps:
                cp.wait()

            def point_body(p, carry2):
                ws = [
                    jnp.broadcast_to(
                        w_v[pl.ds((k * 4 + c) * _CH + p, 16)][0], (16,))
                    for c in range(4)
                ]
                for cc in range(_C // 16):
                    sl = pl.ds(cc * 16, 16)
                    acc = ws[0] * b0[p, sl]
                    acc = acc + ws[1] * b1[p, sl]
                    acc = acc + ws[2] * b2[p, sl]
                    acc = acc + ws[3] * b3[p, sl]
                    o_v[p, sl] = acc
                return carry2

            lax.fori_loop(0, _CH, point_body, 0, unroll=False)
            pltpu.sync_copy(o_v, out_hbm.at[pl.ds(off, _CH)])
            return carry

        lax.fori_loop(0, _NCHUNK, chunk_body, 0, unroll=False)

    return sampler


@functools.cache
def _get_sampler():
    return _make_sampler()


def _sampler(table, idx, w):
    return _get_sampler()(table, idx, w)


def _loss_body(f1_ref, f2_ref, x1_ref, y1_ref, x2_ref, y2_ref, out_ref):
    i = pl.program_id(0)
    f1 = f1_ref[0]
    f2 = f2_ref[0]
    num = lax.dot_general(
        f1, f2, (((1,), (1,)), ((), ())),
        preferred_element_type=jnp.float32)
    na = jnp.sqrt(jnp.sum(f1 * f1, axis=1))[:, None]
    nb = jnp.sqrt(jnp.sum(f2 * f2, axis=1))[None, :]
    thr = _SIM_THR * jnp.maximum(na * nb, 1e-8)
    mask = num >= thr

    x1 = x1_ref[0, 0]
    y1 = y1_ref[0, 0]
    x2 = x2_ref[0, 0]
    y2 = y2_ref[0, 0]
    dx = x1[:, None] - x2[None, :]
    dy = y1[:, None] - y2[None, :]
    dist = jnp.sqrt(dx * dx + dy * dy)

    maskf = mask.astype(jnp.float32)
    cnt = jnp.sum(maskf, axis=1)
    d1 = jnp.min(jnp.where(mask, dist, 1e10), axis=1)
    d2 = jnp.sum(dist * maskf, axis=1) / jnp.maximum(cnt, 1.0)
    m = (jnp.sum(x2) + jnp.sum(y2) >= 0).astype(jnp.float32)
    cost = jnp.where(cnt > 0, (d1 + d2) * (0.5 * m), 0.0)
    obj = jnp.sum(cost)

    @pl.when(i == 0)
    def _():
        out_ref[0, 0] = 0.0

    out_ref[0, 0] += obj

    @pl.when(i == _N_OBJ - 1)
    def _():
        out_ref[0, 0] *= _LOSS_WEIGHT / (_N_P * _N_OBJ)


def _loss(f1_all, f2_all, x1, y1, x2, y2):
    out = pl.pallas_call(
        _loss_body,
        grid=(_N_OBJ,),
        in_specs=[
            pl.BlockSpec((1, _N_P, _C), lambda i: (i, 0, 0)),
            pl.BlockSpec((1, _P2, _C), lambda i: (i, 0, 0)),
            pl.BlockSpec((1, 1, _N_P), lambda i: (i, 0, 0)),
            pl.BlockSpec((1, 1, _N_P), lambda i: (i, 0, 0)),
            pl.BlockSpec((1, 1, _P2), lambda i: (i, 0, 0)),
            pl.BlockSpec((1, 1, _P2), lambda i: (i, 0, 0)),
        ],
        out_specs=pl.BlockSpec((1, 1), lambda i: (0, 0),
                               memory_space=pltpu.SMEM),
        out_shape=jax.ShapeDtypeStruct((1, 1), jnp.float32),
    )(f1_all, f2_all, x1, y1, x2, y2)
    return out[0, 0]


def kernel(point_set_1, point_set_2, feats, key_points):
    table = feats[0].transpose(1, 2, 0).reshape(_H * _W, _C)

    pts = jnp.concatenate(
        [key_points / _IMG, point_set_2.reshape(-1, 2) / _IMG], axis=0)
    ix = pts[:, 0] * _W - 0.5
    iy = pts[:, 1] * _H - 0.5
    x0 = jnp.floor(ix)
    y0 = jnp.floor(iy)
    wx1 = ix - x0
    wx0 = 1.0 - wx1
    wy1 = iy - y0
    wy0 = 1.0 - wy1

    idxs = []
    ws = []
    for xi, yi, wgt in ((x0, y0, wx0 * wy0), (x0 + 1.0, y0, wx1 * wy0),
                        (x0, y0 + 1.0, wx0 * wy1), (x0 + 1.0, y0 + 1.0, wx1 * wy1)):
        valid = ((xi >= 0) & (xi <= _W - 1) & (yi >= 0) & (yi <= _H - 1))
        xc = jnp.clip(xi, 0, _W - 1).astype(jnp.int32)
        yc = jnp.clip(yi, 0, _H - 1).astype(jnp.int32)
        idxs.append(yc * _W + xc)
        ws.append(wgt * valid.astype(jnp.float32))
    idx = (jnp.stack(idxs, axis=0)
           .reshape(4, _NW, _NCHUNK, _CH)
           .transpose(1, 2, 0, 3)
           .reshape(_NW, _NCHUNK * 4, _CH))
    w = (jnp.stack(ws, axis=0)
         .reshape(4, _NW, _NCHUNK, _CH)
         .transpose(1, 2, 0, 3)
         .reshape(_NW, 1, _NCHUNK * 4 * _CH))

    sampled = _sampler(table, idx, w)
    f1_all = sampled[:_N_KP].reshape(_N_OBJ, _N_P, _C)
    f2_all = sampled[_N_KP:].reshape(_N_OBJ, _P2, _C)

    x1 = point_set_1[..., 0].reshape(_N_OBJ, 1, _N_P)
    y1 = point_set_1[..., 1].reshape(_N_OBJ, 1, _N_P)
    x2 = point_set_2[..., 0].reshape(_N_OBJ, 1, _P2)
    y2 = point_set_2[..., 1].reshape(_N_OBJ, 1, _P2)

    return _loss(f1_all, f2_all, x1, y1, x2, y2)


# speedup vs baseline: 2.1046x; 1.2870x over previous
import functools

import jax
import jax.numpy as jnp
from jax import lax
from jax.experimental import pallas as pl
from jax.experimental.pallas import tpu as pltpu
from jax.experimental.pallas import tpu_sc as plsc

_N_OBJ = 8
_N_P = 64
_P2 = 4096
_C = 256
_H = 128
_W = 128
_IMG = 512.0
_SIM_THR = 0.5
_LOSS_WEIGHT = 1.0

_N_KP = _N_OBJ * _N_P
_P_TOTAL = _N_KP + _N_OBJ * _P2
_NW = 32
_PPW = _P_TOTAL // _NW
_CH = 40
_NCHUNK = _PPW // _CH


def _make_sampler():
    mesh = plsc.VectorSubcoreMesh(core_axis_name="c", subcore_axis_name="s")

    @functools.partial(
        pl.kernel,
        mesh=mesh,
        out_type=jax.ShapeDtypeStruct((_P_TOTAL, _C), jnp.float32),
        scratch_types=[
            pltpu.VMEM((_NCHUNK * 4, _CH), jnp.int32),
            pltpu.VMEM((_NCHUNK * 4 * _CH + 16,), jnp.float32),
            pltpu.VMEM((_CH, _C), jnp.float32),
            pltpu.VMEM((_CH, _C), jnp.float32),
            pltpu.VMEM((_CH, _C), jnp.float32),
            pltpu.VMEM((_CH, _C), jnp.float32),
            pltpu.VMEM((_CH, _C), jnp.float32),
            pltpu.VMEM((_CH, _C), jnp.float32),
            pltpu.VMEM((_CH, _C), jnp.float32),
            pltpu.VMEM((_CH, _C), jnp.float32),
            pltpu.VMEM((_CH, _C), jnp.float32),
            pltpu.VMEM((_CH, _C), jnp.float32),
            pltpu.SemaphoreType.DMA,
            pltpu.SemaphoreType.DMA,
            pltpu.SemaphoreType.DMA,
            pltpu.SemaphoreType.DMA,
        ],
    )
    def sampler(table_hbm, idx_hbm, w_hbm, out_hbm,
                idx_v, w_v, a0, a1, a2, a3, c0, c1, c2, c3, oa, ob,
                gsa, gsb, osa, osb):
        wid = lax.axis_index("s") * 2 + lax.axis_index("c")
        base = wid * _PPW
        pltpu.sync_copy(idx_hbm.at[wid], idx_v)
        pltpu.sync_copy(w_hbm.at[wid, 0], w_v.at[pl.ds(0, _NCHUNK * 4 * _CH)])

        bufs = ((a0, a1, a2, a3), (c0, c1, c2, c3))
        obufs = (oa, ob)
        gsems = (gsa, gsb)
        osems = (osa, osb)

        def fire(k, s):
            for c in range(4):
                pltpu.async_copy(
                    table_hbm.at[idx_v.at[k * 4 + c]], bufs[s][c], gsems[s])

        def wait_gather(s):
            for c in range(4):
                pltpu.make_async_copy(
                    table_hbm.at[idx_v.at[c]], bufs[s][c], gsems[s]).wait()

        fire(0, 0)

        def outer(j, carry):
            for b in range(2):
                k = j * 2 + b
                wait_gather(b)

                @pl.when(k + 1 < _NCHUNK)
                def _():
                    fire(k + 1, 1 - b)

                @pl.when(k >= 2)
                def _():
                    pltpu.make_async_copy(
                        obufs[b], out_hbm.at[pl.ds(base, _CH)],
                        osems[b]).wait()

                o_v = obufs[b]
                b0, b1, b2, b3 = bufs[b]

                def point_body(p, carry2):
                    ws = [
                        jnp.broadcast_to(
                            w_v[pl.ds((k * 4 + c) * _CH + p, 16)][0], (16,))
                        for c in range(4)
                    ]
                    for cc in range(_C // 16):
                        sl = pl.ds(cc * 16, 16)
                        acc = ws[0] * b0[p, sl]
                        acc = acc + ws[1] * b1[p, sl]
                        acc = acc + ws[2] * b2[p, sl]
                        acc = acc + ws[3] * b3[p, sl]
                        o_v[p, sl] = acc
                    return carry2

                lax.fori_loop(0, _CH, point_body, 0, unroll=False)
                pltpu.async_copy(
                    o_v, out_hbm.at[pl.ds(base + k * _CH, _CH)], osems[b])
            return carry

        lax.fori_loop(0, _NCHUNK // 2, outer, 0, unroll=False)
        for b in range(2):
            pltpu.make_async_copy(
                obufs[b], out_hbm.at[pl.ds(base, _CH)], osems[b]).wait()

    return sampler


@functools.cache
def _get_sampler():
    return _make_sampler()


def _sampler(table, idx, w):
    return _get_sampler()(table, idx, w)


def _loss_body(f1_ref, f2_ref, x1_ref, y1_ref, x2_ref, y2_ref, out_ref):
    i = pl.program_id(0)
    f1 = f1_ref[0]
    f2 = f2_ref[0]
    num = lax.dot_general(
        f1, f2, (((1,), (1,)), ((), ())),
        preferred_element_type=jnp.float32)
    na = jnp.sqrt(jnp.sum(f1 * f1, axis=1))[:, None]
    nb = jnp.sqrt(jnp.sum(f2 * f2, axis=1))[None, :]
    thr = _SIM_THR * jnp.maximum(na * nb, 1e-8)
    mask = num >= thr

    x1 = x1_ref[0, 0]
    y1 = y1_ref[0, 0]
    x2 = x2_ref[0, 0]
    y2 = y2_ref[0, 0]
    dx = x1[:, None] - x2[None, :]
    dy = y1[:, None] - y2[None, :]
    dist = jnp.sqrt(dx * dx + dy * dy)

    maskf = mask.astype(jnp.float32)
    cnt = jnp.sum(maskf, axis=1)
    d1 = jnp.min(jnp.where(mask, dist, 1e10), axis=1)
    d2 = jnp.sum(dist * maskf, axis=1) / jnp.maximum(cnt, 1.0)
    m = (jnp.sum(x2) + jnp.sum(y2) >= 0).astype(jnp.float32)
    cost = jnp.where(cnt > 0, (d1 + d2) * (0.5 * m), 0.0)
    obj = jnp.sum(cost)

    @pl.when(i == 0)
    def _():
        out_ref[0, 0] = 0.0

    out_ref[0, 0] += obj

    @pl.when(i == _N_OBJ - 1)
    def _():
        out_ref[0, 0] *= _LOSS_WEIGHT / (_N_P * _N_OBJ)


def _loss(f1_all, f2_all, x1, y1, x2, y2):
    out = pl.pallas_call(
        _loss_body,
        grid=(_N_OBJ,),
        in_specs=[
            pl.BlockSpec((1, _N_P, _C), lambda i: (i, 0, 0)),
            pl.BlockSpec((1, _P2, _C), lambda i: (i, 0, 0)),
            pl.BlockSpec((1, 1, _N_P), lambda i: (i, 0, 0)),
            pl.BlockSpec((1, 1, _N_P), lambda i: (i, 0, 0)),
            pl.BlockSpec((1, 1, _P2), lambda i: (i, 0, 0)),
            pl.BlockSpec((1, 1, _P2), lambda i: (i, 0, 0)),
        ],
        out_specs=pl.BlockSpec((1, 1), lambda i: (0, 0),
                               memory_space=pltpu.SMEM),
        out_shape=jax.ShapeDtypeStruct((1, 1), jnp.float32),
    )(f1_all, f2_all, x1, y1, x2, y2)
    return out[0, 0]


def kernel(point_set_1, point_set_2, feats, key_points):
    table = feats[0].transpose(1, 2, 0).reshape(_H * _W, _C)

    pts = jnp.concatenate(
        [key_points / _IMG, point_set_2.reshape(-1, 2) / _IMG], axis=0)
    ix = pts[:, 0] * _W - 0.5
    iy = pts[:, 1] * _H - 0.5
    x0 = jnp.floor(ix)
    y0 = jnp.floor(iy)
    wx1 = ix - x0
    wx0 = 1.0 - wx1
    wy1 = iy - y0
    wy0 = 1.0 - wy1

    idxs = []
    ws = []
    for xi, yi, wgt in ((x0, y0, wx0 * wy0), (x0 + 1.0, y0, wx1 * wy0),
                        (x0, y0 + 1.0, wx0 * wy1), (x0 + 1.0, y0 + 1.0, wx1 * wy1)):
        valid = ((xi >= 0) & (xi <= _W - 1) & (yi >= 0) & (yi <= _H - 1))
        xc = jnp.clip(xi, 0, _W - 1).astype(jnp.int32)
        yc = jnp.clip(yi, 0, _H - 1).astype(jnp.int32)
        idxs.append(yc * _W + xc)
        ws.append(wgt * valid.astype(jnp.float32))
    idx = (jnp.stack(idxs, axis=0)
           .reshape(4, _NW, _NCHUNK, _CH)
           .transpose(1, 2, 0, 3)
           .reshape(_NW, _NCHUNK * 4, _CH))
    w = (jnp.stack(ws, axis=0)
         .reshape(4, _NW, _NCHUNK, _CH)
         .transpose(1, 2, 0, 3)
         .reshape(_NW, 1, _NCHUNK * 4 * _CH))

    sampled = _sampler(table, idx, w)
    f1_all = sampled[:_N_KP].reshape(_N_OBJ, _N_P, _C)
    f2_all = sampled[_N_KP:].reshape(_N_OBJ, _P2, _C)

    x1 = point_set_1[..., 0].reshape(_N_OBJ, 1, _N_P)
    y1 = point_set_1[..., 1].reshape(_N_OBJ, 1, _N_P)
    x2 = point_set_2[..., 0].reshape(_N_OBJ, 1, _P2)
    y2 = point_set_2[..., 1].reshape(_N_OBJ, 1, _P2)

    return _loss(f1_all, f2_all, x1, y1, x2, y2)


# speedup vs baseline: 2.5136x; 1.1943x over previous
import functools

import jax
import jax.numpy as jnp
from jax import lax
from jax.experimental import pallas as pl
from jax.experimental.pallas import tpu as pltpu
from jax.experimental.pallas import tpu_sc as plsc

_N_OBJ = 8
_N_P = 64
_P2 = 4096
_C = 256
_H = 128
_W = 128
_IMG = 512.0
_SIM_THR = 0.5
_LOSS_WEIGHT = 1.0

_N_KP = _N_OBJ * _N_P
_P_TOTAL = _N_KP + _N_OBJ * _P2
_NW = 32
_PPW = _P_TOTAL // _NW
_CH = 40
_NCHUNK = _PPW // _CH


def _make_sampler():
    mesh = plsc.VectorSubcoreMesh(core_axis_name="c", subcore_axis_name="s")

    @functools.partial(
        pl.kernel,
        mesh=mesh,
        out_type=jax.ShapeDtypeStruct((_P_TOTAL, _C), jnp.float32),
        scratch_types=[
            pltpu.VMEM((_NCHUNK * 4, _CH), jnp.int32),
            pltpu.VMEM((_NCHUNK * 4 * _CH + 16,), jnp.float32),
            pltpu.VMEM((_CH, _C), jnp.float32),
            pltpu.VMEM((_CH, _C), jnp.float32),
            pltpu.VMEM((_CH, _C), jnp.float32),
            pltpu.VMEM((_CH, _C), jnp.float32),
            pltpu.VMEM((_CH, _C), jnp.float32),
            pltpu.VMEM((_CH, _C), jnp.float32),
            pltpu.VMEM((_CH, _C), jnp.float32),
            pltpu.VMEM((_CH, _C), jnp.float32),
            pltpu.VMEM((_CH, _C), jnp.float32),
            pltpu.VMEM((_CH, _C), jnp.float32),
            pltpu.SemaphoreType.DMA,
            pltpu.SemaphoreType.DMA,
            pltpu.SemaphoreType.DMA,
            pltpu.SemaphoreType.DMA,
        ],
    )
    def sampler(table_hbm, idx_hbm, w_hbm, out_hbm,
                idx_v, w_v, a0, a1, a2, a3, c0, c1, c2, c3, oa, ob,
                gsa, gsb, osa, osb):
        wid = lax.axis_index("s") * 2 + lax.axis_index("c")
        base = wid * _PPW
        pltpu.sync_copy(idx_hbm.at[wid], idx_v)
        pltpu.sync_copy(w_hbm.at[wid, 0], w_v.at[pl.ds(0, _NCHUNK * 4 * _CH)])

        bufs = ((a0, a1, a2, a3), (c0, c1, c2, c3))
        obufs = (oa, ob)
        gsems = (gsa, gsb)
        osems = (osa, osb)

        def fire(k, s):
            for c in range(4):
                pltpu.async_copy(
                    table_hbm.at[idx_v.at[k * 4 + c]], bufs[s][c], gsems[s])

        def wait_gather(s):
            for c in range(4):
                pltpu.make_async_copy(
                    table_hbm.at[idx_v.at[c]], bufs[s][c], gsems[s]).wait()

        fire(0, 0)

        def outer(j, carry):
            for b in range(2):
                k = j * 2 + b
                wait_gather(b)

                @pl.when(k + 1 < _NCHUNK)
                def _():
                    fire(k + 1, 1 - b)

                @pl.when(k >= 2)
                def _():
                    pltpu.make_async_copy(
                        obufs[b], out_hbm.at[pl.ds(base, _CH)],
                        osems[b]).wait()

                o_v = obufs[b]
                b0, b1, b2, b3 = bufs[b]

                def point_body(p, carry2):
                    ws = [
                        jnp.broadcast_to(
                            w_v[pl.ds((k * 4 + c) * _CH + p, 16)][0], (16,))
                        for c in range(4)
                    ]
                    for cc in range(_C // 16):
                        sl = pl.ds(cc * 16, 16)
                        acc = ws[0] * b0[p, sl]
                        acc = acc + ws[1] * b1[p, sl]
                        acc = acc + ws[2] * b2[p, sl]
                        acc = acc + ws[3] * b3[p, sl]
                        o_v[p, sl] = acc
                    return carry2

                lax.fori_loop(0, _CH, point_body, 0, unroll=2)
                pltpu.async_copy(
                    o_v, out_hbm.at[pl.ds(base + k * _CH, _CH)], osems[b])
            return carry

        lax.fori_loop(0, _NCHUNK // 2, outer, 0, unroll=False)
        for b in range(2):
            pltpu.make_async_copy(
                obufs[b], out_hbm.at[pl.ds(base, _CH)], osems[b]).wait()

    return sampler


@functools.cache
def _get_sampler():
    return _make_sampler()


def _sampler(table, idx, w):
    return _get_sampler()(table, idx, w)


def _loss_body(f1_ref, f2_ref, x1_ref, y1_ref, x2_ref, y2_ref, out_ref):
    i = pl.program_id(0)
    f1 = f1_ref[...]
    f2 = f2_ref[...]
    num = lax.dot_general(
        f1, f2, (((1,), (1,)), ((), ())),
        preferred_element_type=jnp.float32)
    na = jnp.sqrt(jnp.sum(f1 * f1, axis=1))[:, None]
    nb = jnp.sqrt(jnp.sum(f2 * f2, axis=1))[None, :]
    thr = _SIM_THR * jnp.maximum(na * nb, 1e-8)
    mask = num >= thr

    x1 = x1_ref[0, 0]
    y1 = y1_ref[0, 0]
    x2 = x2_ref[0, 0]
    y2 = y2_ref[0, 0]
    dx = x1[:, None] - x2[None, :]
    dy = y1[:, None] - y2[None, :]
    dist = jnp.sqrt(dx * dx + dy * dy)

    maskf = mask.astype(jnp.float32)
    cnt = jnp.sum(maskf, axis=1)
    d1 = jnp.min(jnp.where(mask, dist, 1e10), axis=1)
    d2 = jnp.sum(dist * maskf, axis=1) / jnp.maximum(cnt, 1.0)
    m = (jnp.sum(x2) + jnp.sum(y2) >= 0).astype(jnp.float32)
    cost = jnp.where(cnt > 0, (d1 + d2) * (0.5 * m), 0.0)
    obj = jnp.sum(cost)

    @pl.when(i == 0)
    def _():
        out_ref[0, 0] = 0.0

    out_ref[0, 0] += obj

    @pl.when(i == _N_OBJ - 1)
    def _():
        out_ref[0, 0] *= _LOSS_WEIGHT / (_N_P * _N_OBJ)


def _loss(sampled, x1, y1, x2, y2):
    out = pl.pallas_call(
        _loss_body,
        grid=(_N_OBJ,),
        in_specs=[
            pl.BlockSpec((_N_P, _C), lambda i: (_N_OBJ * _P2 // _N_P + i, 0)),
            pl.BlockSpec((_P2, _C), lambda i: (i, 0)),
            pl.BlockSpec((1, 1, _N_P), lambda i: (i, 0, 0)),
            pl.BlockSpec((1, 1, _N_P), lambda i: (i, 0, 0)),
            pl.BlockSpec((1, 1, _P2), lambda i: (i, 0, 0)),
            pl.BlockSpec((1, 1, _P2), lambda i: (i, 0, 0)),
        ],
        out_specs=pl.BlockSpec((1, 1), lambda i: (0, 0),
                               memory_space=pltpu.SMEM),
        out_shape=jax.ShapeDtypeStruct((1, 1), jnp.float32),
    )(sampled, sampled, x1, y1, x2, y2)
    return out[0, 0]


def kernel(point_set_1, point_set_2, feats, key_points):
    table = feats[0].transpose(1, 2, 0).reshape(_H * _W, _C)

    pts = jnp.concatenate(
        [point_set_2.reshape(-1, 2) / _IMG, key_points / _IMG], axis=0)
    ix = pts[:, 0] * _W - 0.5
    iy = pts[:, 1] * _H - 0.5
    x0 = jnp.floor(ix)
    y0 = jnp.floor(iy)
    wx1 = ix - x0
    wx0 = 1.0 - wx1
    wy1 = iy - y0
    wy0 = 1.0 - wy1

    idxs = []
    ws = []
    for xi, yi, wgt in ((x0, y0, wx0 * wy0), (x0 + 1.0, y0, wx1 * wy0),
                        (x0, y0 + 1.0, wx0 * wy1), (x0 + 1.0, y0 + 1.0, wx1 * wy1)):
        valid = ((xi >= 0) & (xi <= _W - 1) & (yi >= 0) & (yi <= _H - 1))
        xc = jnp.clip(xi, 0, _W - 1).astype(jnp.int32)
        yc = jnp.clip(yi, 0, _H - 1).astype(jnp.int32)
        idxs.append(yc * _W + xc)
        ws.append(wgt * valid.astype(jnp.float32))
    idx = (jnp.stack(idxs, axis=0)
           .reshape(4, _NW, _NCHUNK, _CH)
           .transpose(1, 2, 0, 3)
           .reshape(_NW, _NCHUNK * 4, _CH))
    w = (jnp.stack(ws, axis=0)
         .reshape(4, _NW, _NCHUNK, _CH)
         .transpose(1, 2, 0, 3)
         .reshape(_NW, 1, _NCHUNK * 4 * _CH))

    sampled = _sampler(table, idx, w)

    x1 = point_set_1[..., 0].reshape(_N_OBJ, 1, _N_P)
    y1 = point_set_1[..., 1].reshape(_N_OBJ, 1, _N_P)
    x2 = point_set_2[..., 0].reshape(_N_OBJ, 1, _P2)
    y2 = point_set_2[..., 1].reshape(_N_OBJ, 1, _P2)

    return _loss(sampled, x1, y1, x2, y2)
